# baseline (speedup 1.0000x reference)
import functools

import jax
import jax.numpy as jnp
from jax import lax
from jax.experimental import pallas as pl
from jax.experimental.pallas import tpu as pltpu
from jax.experimental.pallas import tpu_sc as plsc

_B, _L, _C = 4, 8192, 3
_R = 3 * _B
_REL_MIN = 0.15
_EPS = 1e-8
_Q_CONF = 0.7
_Q_CON = 0.3
_MAX_FINITE_BITS = 0x7F7FFFFF
_INF_BITS = 0x7F800000

_NC, _NS = 2, 16


def _prep_body(post_ref, scores_ref, refs_ref, dpinf_ref, dbits_ref, aux_ref):
    p0 = post_ref[0]
    p1 = post_ref[1]
    p2 = post_ref[2]
    h = -(
        (p0 * jnp.log(p0 + _EPS) + p1 * jnp.log(p1 + _EPS))
        + p2 * jnp.log(p2 + _EPS)
    )
    hmax = jnp.log(jnp.float32(_C))
    rel = 1.0 - jnp.clip(h / jnp.maximum(hmax, 1e-8), 0.0, 1.0)
    valid = rel >= _REL_MIN
    d = jnp.abs(scores_ref[...] - refs_ref[...])
    dpinf = jnp.where(valid, d, jnp.inf)
    dpinf_ref[...] = dpinf
    dbits_ref[...] = lax.bitcast_convert_type(dpinf, jnp.int32)

    nf = jnp.sum(jnp.where(valid, 1.0, 0.0), axis=1, keepdims=True)

    def rank_of(q):
        pos = jnp.float32(q) * (nf - 1.0)
        lo = jnp.clip(jnp.floor(pos).astype(jnp.int32), 0, _L - 1)
        hi = jnp.clip(jnp.ceil(pos).astype(jnp.int32), 0, _L - 1)
        return lo, hi

    lo1, hi1 = rank_of(_Q_CONF)
    lo2, hi2 = rank_of(_Q_CON)
    n = nf.astype(jnp.int32)
    kmed = jnp.clip(lax.shift_right_arithmetic(n - 1, 1), 0, _L - 1)

    lanes = lax.broadcasted_iota(jnp.int32, (_R, 128), 1)
    aux = jnp.where(
        lanes == 0,
        lo1,
        jnp.where(
            lanes == 1,
            hi1,
            jnp.where(
                lanes == 2,
                lo2,
                jnp.where(lanes == 3, hi2, jnp.where(lanes == 4, kmed, 0)),
            ),
        ),
    )
    aux_ref[...] = aux


_prep = pl.pallas_call(
    _prep_body,
    out_shape=(
        jax.ShapeDtypeStruct((_R, _L), jnp.float32),
        jax.ShapeDtypeStruct((_R, _L), jnp.int32),
        jax.ShapeDtypeStruct((_R, 128), jnp.int32),
    ),
)


def _sc_select_body(dbits_hbm, aux_hbm, out_hbm, db_v, aux_v, sum_v, out_v):
    wid = lax.axis_index("s") * _NC + lax.axis_index("c")

    @pl.when(wid < _R)
    def _():
        pltpu.sync_copy(dbits_hbm.at[wid], db_v)
        pltpu.sync_copy(aux_hbm.at[wid], aux_v)

        aux0 = aux_v[pl.ds(0, 16)]
        ks = [aux0[i] for i in range(5)]

        ones = jnp.full((16,), 1, jnp.int32)
        zeros = jnp.full((16,), 0, jnp.int32)

        def counts_le(thrs):
            nt = len(thrs)

            def body(j, accs):
                base = j * 256
                new = list(accs)
                for u in range(16):
                    bits = db_v[pl.ds(base + u * 16, 16)]
                    for t in range(nt):
                        new[t] = new[t] + jnp.where(
                            bits <= thrs[t], ones, zeros
                        )
                return tuple(new)

            accs = plsc.parallel_loop(
                0, _L // 256, carry=tuple(zeros for _ in range(nt))
            )(body)
            cnts = []
            for acc in accs:
                c = acc[0]
                for i in range(1, 16):
                    c = c + acc[i]
                cnts.append(c)
            return cnts

        def search_body(_, carry):
            los, his = carry
            mids = [
                lo + lax.shift_right_logical(hi - lo, 1)
                for lo, hi in zip(los, his)
            ]
            cnts = counts_le(mids)
            nlos, nhis = [], []
            for lo, hi, mid, cnt, k in zip(los, his, mids, cnts, ks):
                ge = cnt >= k + 1
                nhis.append(jnp.where(ge, mid, hi))
                nlos.append(jnp.where(ge, lo, mid + 1))
            return tuple(nlos), tuple(nhis)

        los, _his = lax.fori_loop(
            0,
            31,
            search_body,
            (
                tuple(jnp.int32(0) for _ in ks),
                tuple(jnp.int32(_INF_BITS) for _ in ks),
            ),
        )

        lanes = jnp.arange(16, dtype=jnp.int32)
        vec = jnp.where(
            lanes == 0,
            los[0],
            jnp.where(
                lanes == 1,
                los[1],
                jnp.where(
                    lanes == 2,
                    los[2],
                    jnp.where(lanes == 3, los[3], los[4]),
                ),
            ),
        )
        out_v[...] = vec
        pltpu.sync_copy(out_v, out_hbm.at[wid])


@functools.cache
def _sc_select():
    return pl.kernel(
        _sc_select_body,
        out_type=jax.ShapeDtypeStruct((_R, 16), jnp.int32),
        mesh=plsc.VectorSubcoreMesh(
            core_axis_name="c", subcore_axis_name="s", num_cores=_NC,
            num_subcores=_NS,
        ),
        scratch_types=[
            pltpu.VMEM((_L,), jnp.int32),
            pltpu.VMEM((128,), jnp.int32),
            pltpu.VMEM((16,), jnp.int32),
            pltpu.VMEM((16,), jnp.int32),
        ],
    )


def _finalize_body(dpinf_ref, sel_ref, scores_ref, refs_ref, con_ref, conf_ref):
    dp = dpinf_ref[...]
    valid = dp < jnp.inf
    d = jnp.abs(scores_ref[...] - refs_ref[...])
    dn = jnp.where(valid, d, -jnp.inf)

    def sbit(i):
        return lax.bitcast_convert_type(sel_ref[:, i : i + 1], jnp.float32)

    s_lo1, s_hi1 = sbit(0), sbit(1)
    s_lo2, s_hi2 = sbit(2), sbit(3)
    med = sbit(4)
    nf = jnp.sum(jnp.where(valid, 1.0, 0.0), axis=1, keepdims=True)

    def frac_of(q):
        pos = jnp.float32(q) * (nf - 1.0)
        lo = jnp.clip(jnp.floor(pos).astype(jnp.int32), 0, _L - 1)
        return pos - lo.astype(jnp.float32)

    thr_conf = s_lo1 + frac_of(_Q_CONF) * (s_hi1 - s_lo1)
    thr_con = s_lo2 + frac_of(_Q_CON) * (s_hi2 - s_lo2)

    conf0 = dn >= thr_conf
    con0 = dp <= thr_con
    ov = conf0 & con0
    gt = d > med
    con1 = con0 & jnp.logical_not(ov & gt)
    conf1 = conf0 & jnp.logical_not(ov & jnp.logical_not(gt))
    con1f = jnp.where(con1, 1.0, 0.0)
    conf1f = jnp.where(conf1, 1.0, 0.0)

    cnt_conf = jnp.sum(conf1f, axis=1, keepdims=True)
    cnt_con = jnp.sum(con1f, axis=1, keepdims=True)

    iota = lax.broadcasted_iota(jnp.int32, (_R, _L), 1)
    dmax = jnp.max(d, axis=1, keepdims=True)
    amax = jnp.min(
        jnp.where(d == dmax, iota, _L), axis=1, keepdims=True
    )
    dmin = jnp.min(d, axis=1, keepdims=True)
    amin = jnp.min(
        jnp.where(d == dmin, iota, _L), axis=1, keepdims=True
    )
    onehot_amax = jnp.where(iota == amax, 1.0, 0.0)
    onehot_amin = jnp.where(iota == amin, 1.0, 0.0)
    onehot0 = jnp.where(iota == 0, 1.0, 0.0)
    onehot1 = jnp.where(iota == 1, 1.0, 0.0)

    conf2f = jnp.where(cnt_conf == 0.0, onehot_amax, conf1f)
    con2f = jnp.where(cnt_con == 0.0, onehot_amin, con1f)

    empty = nf == 0.0
    conf3f = jnp.where(empty, onehot0, conf2f)
    con3f = jnp.where(empty, onehot1, con2f)

    con_ref[...] = con3f
    conf_ref[...] = conf3f


_finalize = pl.pallas_call(
    _finalize_body,
    out_shape=(
        jax.ShapeDtypeStruct((_R, _L), jnp.float32),
        jax.ShapeDtypeStruct((_R, _L), jnp.float32),
    ),
)


def kernel(
    posteriors_T,
    posteriors_A,
    posteriors_V,
    senti_scores_T,
    senti_scores_A,
    senti_scores_V,
    senti_ref_T,
    senti_ref_A,
    senti_ref_V,
):
    post = jnp.stack([posteriors_T, posteriors_A, posteriors_V], axis=0)
    post = post.transpose(3, 0, 1, 2).reshape(_C, _R, _L)
    scores = jnp.stack(
        [senti_scores_T, senti_scores_A, senti_scores_V], axis=0
    ).reshape(_R, _L)
    refs = jnp.stack([senti_ref_T, senti_ref_A, senti_ref_V], axis=0).reshape(
        _R, _L
    )

    dpinf, dbits, aux = _prep(post, scores, refs)
    sel = _sc_select()(dbits, aux)
    con, conf = _finalize(dpinf, sel, scores, refs)

    con = con.astype(jnp.bool_).reshape(3, _B, _L)
    conf = conf.astype(jnp.bool_).reshape(3, _B, _L)
    return (con[0], con[1], con[2], conf[0], conf[1], conf[2])


# speedup vs baseline: 4.8426x; 4.8426x over previous
import functools

import jax
import jax.numpy as jnp
from jax import lax
from jax.experimental import pallas as pl
from jax.experimental.pallas import tpu as pltpu
from jax.experimental.pallas import tpu_sc as plsc

_B, _L, _C = 4, 8192, 3
_R = 3 * _B
_REL_MIN = 0.15
_EPS = 1e-8
_Q_CONF = 0.7
_Q_CON = 0.3
_MAX_FINITE_BITS = 0x7F7FFFFF
_INF_BITS = 0x7F800000

_NC, _NS = 2, 16


def _prep_body(post_ref, scores_ref, refs_ref, dpinf_ref, dbits_ref, aux_ref):
    p0 = post_ref[0]
    p1 = post_ref[1]
    p2 = post_ref[2]
    h = -(
        (p0 * jnp.log(p0 + _EPS) + p1 * jnp.log(p1 + _EPS))
        + p2 * jnp.log(p2 + _EPS)
    )
    hmax = jnp.log(jnp.float32(_C))
    rel = 1.0 - jnp.clip(h / jnp.maximum(hmax, 1e-8), 0.0, 1.0)
    valid = rel >= _REL_MIN
    d = jnp.abs(scores_ref[...] - refs_ref[...])
    dpinf = jnp.where(valid, d, jnp.inf)
    dpinf_ref[...] = dpinf
    dbits_ref[...] = lax.bitcast_convert_type(dpinf, jnp.int32)

    nf = jnp.sum(jnp.where(valid, 1.0, 0.0), axis=1, keepdims=True)

    def rank_of(q):
        pos = jnp.float32(q) * (nf - 1.0)
        lo = jnp.clip(jnp.floor(pos).astype(jnp.int32), 0, _L - 1)
        hi = jnp.clip(jnp.ceil(pos).astype(jnp.int32), 0, _L - 1)
        return lo, hi

    lo1, hi1 = rank_of(_Q_CONF)
    lo2, hi2 = rank_of(_Q_CON)
    n = nf.astype(jnp.int32)
    kmed = jnp.clip(lax.shift_right_arithmetic(n - 1, 1), 0, _L - 1)

    lanes = lax.broadcasted_iota(jnp.int32, (_R, 128), 1)
    aux = jnp.where(
        lanes == 0,
        lo1,
        jnp.where(
            lanes == 1,
            hi1,
            jnp.where(
                lanes == 2,
                lo2,
                jnp.where(lanes == 3, hi2, jnp.where(lanes == 4, kmed, 0)),
            ),
        ),
    )
    aux_ref[...] = aux


_prep = pl.pallas_call(
    _prep_body,
    out_shape=(
        jax.ShapeDtypeStruct((_R, _L), jnp.float32),
        jax.ShapeDtypeStruct((_R, _L), jnp.int32),
        jax.ShapeDtypeStruct((_R, 128), jnp.int32),
    ),
)


def _sc_select_body(dbits_hbm, aux_hbm, out_hbm, db_v, aux_v, out_v):
    wid = lax.axis_index("s") * _NC + lax.axis_index("c")

    ones = jnp.full((16,), 1, jnp.int32)
    zeros = jnp.full((16,), 0, jnp.int32)

    def do_unit(u):
        row = u // 5
        slot = u - row * 5

        pltpu.sync_copy(aux_hbm.at[row], aux_v)
        a = aux_v[pl.ds(0, 16)]
        k = jnp.where(
            slot == 0,
            a[0],
            jnp.where(
                slot == 1,
                a[1],
                jnp.where(slot == 2, a[2], jnp.where(slot == 3, a[3], a[4])),
            ),
        )
        pltpu.sync_copy(dbits_hbm.at[row], db_v)

        def count_le(thr):
            def body(j, accs):
                base = j * 256
                new = list(accs)
                for uu in range(16):
                    bits = db_v[pl.ds(base + uu * 16, 16)]
                    new[uu % 4] = new[uu % 4] + jnp.where(
                        bits <= thr, ones, zeros
                    )
                return tuple(new)

            accs = plsc.parallel_loop(
                0, _L // 256, carry=(zeros, zeros, zeros, zeros)
            )(body)
            acc = (accs[0] + accs[1]) + (accs[2] + accs[3])
            c = acc[0]
            for i in range(1, 16):
                c = c + acc[i]
            return c

        def search_body(_, carry):
            lo, hi = carry
            mid = lo + lax.shift_right_logical(hi - lo, 1)
            ge = count_le(mid) >= k + 1
            return jnp.where(ge, lo, mid + 1), jnp.where(ge, mid, hi)

        lo, _hi = lax.fori_loop(
            0, 31, search_body, (jnp.int32(0), jnp.int32(_INF_BITS))
        )
        out_v[...] = jnp.full((16,), lo, jnp.int32)
        pltpu.sync_copy(out_v, out_hbm.at[u])

    do_unit(wid)

    @pl.when(wid + 32 < _R * 5)
    def _():
        do_unit(wid + 32)


@functools.cache
def _sc_select():
    return pl.kernel(
        _sc_select_body,
        out_type=jax.ShapeDtypeStruct((64, 16), jnp.int32),
        mesh=plsc.VectorSubcoreMesh(
            core_axis_name="c", subcore_axis_name="s", num_cores=_NC,
            num_subcores=_NS,
        ),
        scratch_types=[
            pltpu.VMEM((_L,), jnp.int32),
            pltpu.VMEM((128,), jnp.int32),
            pltpu.VMEM((16,), jnp.int32),
        ],
    )


def _finalize_body(dpinf_ref, sel_ref, scores_ref, refs_ref, con_ref, conf_ref):
    dp = dpinf_ref[...]
    valid = dp < jnp.inf
    d = jnp.abs(scores_ref[...] - refs_ref[...])
    dn = jnp.where(valid, d, -jnp.inf)

    def sbit(i):
        return lax.bitcast_convert_type(sel_ref[:, i : i + 1], jnp.float32)

    s_lo1, s_hi1 = sbit(0), sbit(1)
    s_lo2, s_hi2 = sbit(2), sbit(3)
    med = sbit(4)
    nf = jnp.sum(jnp.where(valid, 1.0, 0.0), axis=1, keepdims=True)

    def frac_of(q):
        pos = jnp.float32(q) * (nf - 1.0)
        lo = jnp.clip(jnp.floor(pos).astype(jnp.int32), 0, _L - 1)
        return pos - lo.astype(jnp.float32)

    thr_conf = s_lo1 + frac_of(_Q_CONF) * (s_hi1 - s_lo1)
    thr_con = s_lo2 + frac_of(_Q_CON) * (s_hi2 - s_lo2)

    conf0 = dn >= thr_conf
    con0 = dp <= thr_con
    ov = conf0 & con0
    gt = d > med
    con1 = con0 & jnp.logical_not(ov & gt)
    conf1 = conf0 & jnp.logical_not(ov & jnp.logical_not(gt))
    con1f = jnp.where(con1, 1.0, 0.0)
    conf1f = jnp.where(conf1, 1.0, 0.0)

    cnt_conf = jnp.sum(conf1f, axis=1, keepdims=True)
    cnt_con = jnp.sum(con1f, axis=1, keepdims=True)

    iota = lax.broadcasted_iota(jnp.int32, (_R, _L), 1)
    dmax = jnp.max(d, axis=1, keepdims=True)
    amax = jnp.min(
        jnp.where(d == dmax, iota, _L), axis=1, keepdims=True
    )
    dmin = jnp.min(d, axis=1, keepdims=True)
    amin = jnp.min(
        jnp.where(d == dmin, iota, _L), axis=1, keepdims=True
    )
    onehot_amax = jnp.where(iota == amax, 1.0, 0.0)
    onehot_amin = jnp.where(iota == amin, 1.0, 0.0)
    onehot0 = jnp.where(iota == 0, 1.0, 0.0)
    onehot1 = jnp.where(iota == 1, 1.0, 0.0)

    conf2f = jnp.where(cnt_conf == 0.0, onehot_amax, conf1f)
    con2f = jnp.where(cnt_con == 0.0, onehot_amin, con1f)

    empty = nf == 0.0
    conf3f = jnp.where(empty, onehot0, conf2f)
    con3f = jnp.where(empty, onehot1, con2f)

    con_ref[...] = con3f
    conf_ref[...] = conf3f


_finalize = pl.pallas_call(
    _finalize_body,
    out_shape=(
        jax.ShapeDtypeStruct((_R, _L), jnp.float32),
        jax.ShapeDtypeStruct((_R, _L), jnp.float32),
    ),
)


def kernel(
    posteriors_T,
    posteriors_A,
    posteriors_V,
    senti_scores_T,
    senti_scores_A,
    senti_scores_V,
    senti_ref_T,
    senti_ref_A,
    senti_ref_V,
):
    post = jnp.stack([posteriors_T, posteriors_A, posteriors_V], axis=0)
    post = post.transpose(3, 0, 1, 2).reshape(_C, _R, _L)
    scores = jnp.stack(
        [senti_scores_T, senti_scores_A, senti_scores_V], axis=0
    ).reshape(_R, _L)
    refs = jnp.stack([senti_ref_T, senti_ref_A, senti_ref_V], axis=0).reshape(
        _R, _L
    )

    dpinf, dbits, aux = _prep(post, scores, refs)
    sel_raw = _sc_select()(dbits, aux)
    sel = sel_raw[: _R * 5, 0].reshape(_R, 5)
    con, conf = _finalize(dpinf, sel, scores, refs)

    con = con.astype(jnp.bool_).reshape(3, _B, _L)
    conf = conf.astype(jnp.bool_).reshape(3, _B, _L)
    return (con[0], con[1], con[2], conf[0], conf[1], conf[2])


# speedup vs baseline: 4.9282x; 1.0177x over previous
import functools

import jax
import jax.numpy as jnp
from jax import lax
from jax.experimental import pallas as pl
from jax.experimental.pallas import tpu as pltpu
from jax.experimental.pallas import tpu_sc as plsc

_B, _L, _C = 4, 8192, 3
_R = 3 * _B
_REL_MIN = 0.15
_EPS = 1e-8
_Q_CONF = 0.7
_Q_CON = 0.3
_MAX_FINITE_BITS = 0x7F7FFFFF
_INF_BITS = 0x7F800000

_NC, _NS = 2, 16


def _prep_body(post_ref, scores_ref, refs_ref, dpinf_ref, dbits_ref, pk_ref, aux_ref):
    p0 = post_ref[0]
    p1 = post_ref[1]
    p2 = post_ref[2]
    h = -(
        (p0 * jnp.log(p0 + _EPS) + p1 * jnp.log(p1 + _EPS))
        + p2 * jnp.log(p2 + _EPS)
    )
    hmax = jnp.log(jnp.float32(_C))
    rel = 1.0 - jnp.clip(h / jnp.maximum(hmax, 1e-8), 0.0, 1.0)
    valid = rel >= _REL_MIN
    d = jnp.abs(scores_ref[...] - refs_ref[...])
    dpinf = jnp.where(valid, d, jnp.inf)
    dpinf_ref[...] = dpinf
    bits = lax.bitcast_convert_type(dpinf, jnp.int32)
    dbits_ref[...] = bits
    bt = lax.shift_right_logical(bits, 16)
    pk_ref[...] = lax.shift_left(bt[:, : _L // 2], 16) | bt[:, _L // 2 :]

    nf = jnp.sum(jnp.where(valid, 1.0, 0.0), axis=1, keepdims=True)

    def rank_of(q):
        pos = jnp.float32(q) * (nf - 1.0)
        lo = jnp.clip(jnp.floor(pos).astype(jnp.int32), 0, _L - 1)
        hi = jnp.clip(jnp.ceil(pos).astype(jnp.int32), 0, _L - 1)
        return lo, hi

    lo1, hi1 = rank_of(_Q_CONF)
    lo2, hi2 = rank_of(_Q_CON)
    n = nf.astype(jnp.int32)
    kmed = jnp.clip(lax.shift_right_arithmetic(n - 1, 1), 0, _L - 1)

    lanes = lax.broadcasted_iota(jnp.int32, (_R, 128), 1)
    aux = jnp.where(
        lanes == 0,
        lo1,
        jnp.where(
            lanes == 1,
            hi1,
            jnp.where(
                lanes == 2,
                lo2,
                jnp.where(lanes == 3, hi2, jnp.where(lanes == 4, kmed, 0)),
            ),
        ),
    )
    aux_ref[...] = aux


_prep = pl.pallas_call(
    _prep_body,
    out_shape=(
        jax.ShapeDtypeStruct((_R, _L), jnp.float32),
        jax.ShapeDtypeStruct((_R, _L), jnp.int32),
        jax.ShapeDtypeStruct((_R, _L // 2), jnp.int32),
        jax.ShapeDtypeStruct((_R, 128), jnp.int32),
    ),
)


def _sc_select_body(dbits_hbm, pk_hbm, aux_hbm, out_hbm, db_v, pk_v, aux_v, out_v):
    wid = lax.axis_index("s") * _NC + lax.axis_index("c")

    ones = jnp.full((16,), 1, jnp.int32)
    zeros = jnp.full((16,), 0, jnp.int32)

    def do_unit(u):
        row = u // 5
        slot = u - row * 5

        pltpu.sync_copy(aux_hbm.at[row], aux_v)
        a = aux_v[pl.ds(0, 16)]
        k = jnp.where(
            slot == 0,
            a[0],
            jnp.where(
                slot == 1,
                a[1],
                jnp.where(slot == 2, a[2], jnp.where(slot == 3, a[3], a[4])),
            ),
        )
        pltpu.sync_copy(dbits_hbm.at[row], db_v)
        pltpu.sync_copy(pk_hbm.at[row], pk_v)

        swar_bias = jnp.int32(-2147450880)
        swar_mask = jnp.int32(0x00010001)

        def count_le16(t):
            s = jnp.full((16,), t * 65537 + swar_bias, jnp.int32)

            def body(j, accs):
                base = j * 256
                new = list(accs)
                for uu in range(16):
                    p = pk_v[pl.ds(base + uu * 16, 16)]
                    y = lax.shift_right_logical(s - p, 15) & swar_mask
                    new[uu % 4] = new[uu % 4] + y
                return tuple(new)

            accs = plsc.parallel_loop(
                0, _L // 512, carry=(zeros, zeros, zeros, zeros)
            )(body)
            accT = (accs[0] + accs[1]) + (accs[2] + accs[3])
            tot = (accT & 0xFFFF) + lax.shift_right_logical(accT, 16)
            c = tot[0]
            for i in range(1, 16):
                c = c + tot[i]
            return c

        def count_le(thr):
            def body(j, accs):
                base = j * 256
                new = list(accs)
                for uu in range(16):
                    bits = db_v[pl.ds(base + uu * 16, 16)]
                    new[uu % 4] = new[uu % 4] + jnp.where(
                        bits <= thr, ones, zeros
                    )
                return tuple(new)

            accs = plsc.parallel_loop(
                0, _L // 256, carry=(zeros, zeros, zeros, zeros)
            )(body)
            acc = (accs[0] + accs[1]) + (accs[2] + accs[3])
            c = acc[0]
            for i in range(1, 16):
                c = c + acc[i]
            return c

        def search16_body(_, carry):
            lo, hi = carry
            mid = lo + lax.shift_right_logical(hi - lo, 1)
            ge = count_le16(mid) >= k + 1
            return jnp.where(ge, lo, mid + 1), jnp.where(ge, mid, hi)

        t16, _ = lax.fori_loop(
            0, 15, search16_body,
            (jnp.int32(0), jnp.int32(_INF_BITS >> 16)),
        )

        def search_body(_, carry):
            lo, hi = carry
            mid = lo + lax.shift_right_logical(hi - lo, 1)
            ge = count_le(mid) >= k + 1
            return jnp.where(ge, lo, mid + 1), jnp.where(ge, mid, hi)

        lo, _hi = lax.fori_loop(
            0, 16, search_body,
            (
                lax.shift_left(t16, 16),
                lax.shift_left(t16, 16) + jnp.int32(0xFFFF),
            ),
        )
        out_v[...] = jnp.full((16,), lo, jnp.int32)
        pltpu.sync_copy(out_v, out_hbm.at[u])

    do_unit(wid)

    @pl.when(wid + 32 < _R * 5)
    def _():
        do_unit(wid + 32)


@functools.cache
def _sc_select():
    return pl.kernel(
        _sc_select_body,
        out_type=jax.ShapeDtypeStruct((64, 16), jnp.int32),
        mesh=plsc.VectorSubcoreMesh(
            core_axis_name="c", subcore_axis_name="s", num_cores=_NC,
            num_subcores=_NS,
        ),
        scratch_types=[
            pltpu.VMEM((_L,), jnp.int32),
            pltpu.VMEM((_L // 2,), jnp.int32),
            pltpu.VMEM((128,), jnp.int32),
            pltpu.VMEM((16,), jnp.int32),
        ],
    )


def _finalize_body(dpinf_ref, sel_ref, scores_ref, refs_ref, con_ref, conf_ref):
    dp = dpinf_ref[...]
    valid = dp < jnp.inf
    d = jnp.abs(scores_ref[...] - refs_ref[...])
    dn = jnp.where(valid, d, -jnp.inf)

    def sbit(i):
        return lax.bitcast_convert_type(sel_ref[:, i : i + 1], jnp.float32)

    s_lo1, s_hi1 = sbit(0), sbit(1)
    s_lo2, s_hi2 = sbit(2), sbit(3)
    med = sbit(4)
    nf = jnp.sum(jnp.where(valid, 1.0, 0.0), axis=1, keepdims=True)

    def frac_of(q):
        pos = jnp.float32(q) * (nf - 1.0)
        lo = jnp.clip(jnp.floor(pos).astype(jnp.int32), 0, _L - 1)
        return pos - lo.astype(jnp.float32)

    thr_conf = s_lo1 + frac_of(_Q_CONF) * (s_hi1 - s_lo1)
    thr_con = s_lo2 + frac_of(_Q_CON) * (s_hi2 - s_lo2)

    conf0 = dn >= thr_conf
    con0 = dp <= thr_con
    ov = conf0 & con0
    gt = d > med
    con1 = con0 & jnp.logical_not(ov & gt)
    conf1 = conf0 & jnp.logical_not(ov & jnp.logical_not(gt))
    con1f = jnp.where(con1, 1.0, 0.0)
    conf1f = jnp.where(conf1, 1.0, 0.0)

    cnt_conf = jnp.sum(conf1f, axis=1, keepdims=True)
    cnt_con = jnp.sum(con1f, axis=1, keepdims=True)

    iota = lax.broadcasted_iota(jnp.int32, (_R, _L), 1)
    dmax = jnp.max(d, axis=1, keepdims=True)
    amax = jnp.min(
        jnp.where(d == dmax, iota, _L), axis=1, keepdims=True
    )
    dmin = jnp.min(d, axis=1, keepdims=True)
    amin = jnp.min(
        jnp.where(d == dmin, iota, _L), axis=1, keepdims=True
    )
    onehot_amax = jnp.where(iota == amax, 1.0, 0.0)
    onehot_amin = jnp.where(iota == amin, 1.0, 0.0)
    onehot0 = jnp.where(iota == 0, 1.0, 0.0)
    onehot1 = jnp.where(iota == 1, 1.0, 0.0)

    conf2f = jnp.where(cnt_conf == 0.0, onehot_amax, conf1f)
    con2f = jnp.where(cnt_con == 0.0, onehot_amin, con1f)

    empty = nf == 0.0
    conf3f = jnp.where(empty, onehot0, conf2f)
    con3f = jnp.where(empty, onehot1, con2f)

    con_ref[...] = con3f
    conf_ref[...] = conf3f


_finalize = pl.pallas_call(
    _finalize_body,
    out_shape=(
        jax.ShapeDtypeStruct((_R, _L), jnp.float32),
        jax.ShapeDtypeStruct((_R, _L), jnp.float32),
    ),
)


def kernel(
    posteriors_T,
    posteriors_A,
    posteriors_V,
    senti_scores_T,
    senti_scores_A,
    senti_scores_V,
    senti_ref_T,
    senti_ref_A,
    senti_ref_V,
):
    post = jnp.stack([posteriors_T, posteriors_A, posteriors_V], axis=0)
    post = post.transpose(3, 0, 1, 2).reshape(_C, _R, _L)
    scores = jnp.stack(
        [senti_scores_T, senti_scores_A, senti_scores_V], axis=0
    ).reshape(_R, _L)
    refs = jnp.stack([senti_ref_T, senti_ref_A, senti_ref_V], axis=0).reshape(
        _R, _L
    )

    dpinf, dbits, pk, aux = _prep(post, scores, refs)
    sel_raw = _sc_select()(dbits, pk, aux)
    sel = sel_raw[: _R * 5, 0].reshape(_R, 5)
    con, conf = _finalize(dpinf, sel, scores, refs)

    con = con.astype(jnp.bool_).reshape(3, _B, _L)
    conf = conf.astype(jnp.bool_).reshape(3, _B, _L)
    return (con[0], con[1], con[2], conf[0], conf[1], conf[2])
